# baseline (speedup 1.0000x reference)
import functools

import jax
import jax.numpy as jnp
from jax import lax
from jax.experimental import pallas as pl
from jax.experimental.pallas import tpu as pltpu
from jax.experimental.pallas import tpu_sc as plsc

NFIELD = 26
SQL_NEMB = 16
DATA_NEMB = 16
K = 16
HID_GATE = 256
HID_MOE = 512
B = 4096
GATE_IN = NFIELD * SQL_NEMB
EXP_IN = NFIELD * DATA_NEMB

_NC = 2
_NS = 16
_NW = _NC * _NS

_ROWS = B * NFIELD
_BPW = _ROWS // _NW

_TILE_B = 1024
_NT = B // _TILE_B


_V1 = 100000
_V2 = 100027
_V2P = 100352
_TCH = 3136


def _sc_transpose(itab_t, stab_t):
    mesh = plsc.VectorSubcoreMesh(core_axis_name="c", subcore_axis_name="s")

    @functools.partial(
        pl.kernel,
        out_type=(
            jax.ShapeDtypeStruct((_V1, DATA_NEMB), jnp.float32),
            jax.ShapeDtypeStruct((_V2P, SQL_NEMB), jnp.float32),
        ),
        mesh=mesh,
        compiler_params=pltpu.CompilerParams(
            use_tc_tiling_on_sc=False, needs_layout_passes=False),
        scratch_types=[
            pltpu.VMEM((16, _TCH), jnp.float32),
            pltpu.VMEM((_TCH, 16), jnp.float32),
        ],
    )
    def transpose_k(it_hbm, st_hbm, io_hbm, so_hbm, chunk_v, out_v):
        wid = lax.axis_index("s") * _NC + lax.axis_index("c")
        lanes = lax.broadcasted_iota(jnp.int32, (16,), 0)

        def one_table(src_hbm, dst_hbm, base):
            pltpu.sync_copy(src_hbm.at[:, pl.ds(base, _TCH)], chunk_v)

            def block(b, _):
                rows = b * 16 + lanes
                for d in range(16):
                    dvec = jnp.full((16,), d, jnp.int32)
                    v = plsc.load_gather(chunk_v, [dvec, rows])
                    plsc.store_scatter(out_v, [rows, dvec], v)
                return _

            lax.fori_loop(0, _TCH // 16, block, 0)
            pltpu.sync_copy(out_v, dst_hbm.at[pl.ds(base, _TCH), :])

        one_table(it_hbm, io_hbm, jnp.minimum(wid * _TCH, _V1 - _TCH))
        one_table(st_hbm, so_hbm, wid * _TCH)

    return transpose_k(itab_t, stab_t)


def _sc_gather(x_idx, sql_idx, input_table, sql_table):
    mesh = plsc.VectorSubcoreMesh(core_axis_name="c", subcore_axis_name="s")

    bw = B // _NW

    @functools.partial(
        pl.kernel,
        out_type=(
            jax.ShapeDtypeStruct((_ROWS, DATA_NEMB), jnp.float32),
            jax.ShapeDtypeStruct((_ROWS, SQL_NEMB), jnp.float32),
        ),
        mesh=mesh,
        compiler_params=pltpu.CompilerParams(use_tc_tiling_on_sc=False),
        scratch_types=[
            pltpu.VMEM((_BPW,), jnp.int32),
            pltpu.VMEM((_BPW, DATA_NEMB), jnp.float32),
            pltpu.SemaphoreType.DMA,
        ],
    )
    def gather_k(xi_hbm, si_hbm, itab_hbm, stab_hbm, xout_hbm, sout_hbm,
                 idx_v, rows_v, sem):
        wid = lax.axis_index("s") * _NC + lax.axis_index("c")
        base = wid * _BPW
        pltpu.sync_copy(xi_hbm.at[pl.ds(base, _BPW)], idx_v)
        pltpu.async_copy(itab_hbm.at[idx_v], rows_v, sem).wait()
        pltpu.sync_copy(rows_v, xout_hbm.at[pl.ds(base, _BPW)])
        pltpu.sync_copy(si_hbm.at[pl.ds(base, _BPW)], idx_v)
        pltpu.async_copy(stab_hbm.at[idx_v], rows_v, sem).wait()
        pltpu.sync_copy(rows_v, sout_hbm.at[pl.ds(base, _BPW)])

    return gather_k(x_idx, sql_idx, input_table, sql_table)


def _tc_body(sql_ref, xf_ref, gw1_ref, gb1_ref, gw2_ref, gb2_ref,
             ew1_ref, eb1_ref, w2sel_ref, eb2_ref,
             y_ref, imp_ref, spa_ref, acc_imp, acc_spa):
    i = pl.program_id(0)
    f32 = jnp.float32
    bf16 = jnp.bfloat16

    sql_t = sql_ref[...]
    gh = jnp.maximum(
        jnp.dot(sql_t, gw1_ref[...], preferred_element_type=f32) + gb1_ref[...],
        0.0)
    logits = (jnp.dot(gh, gw2_ref[...],
                      preferred_element_type=f32) + gb2_ref[...])

    t = logits.shape[0]
    z_t = logits.T
    rowid = lax.broadcasted_iota(jnp.int32, (K, t), 0)
    kz = jnp.zeros((1, t), f32)
    ssum = jnp.zeros((1, t), f32)
    for j in range(K):
        zj = z_t[j:j + 1, :]
        above = jnp.logical_or(
            z_t > zj, jnp.logical_and(z_t == zj, rowid < j))
        r = 1.0 + jnp.sum(above.astype(f32), axis=0, keepdims=True)
        s = zj + jnp.sum(jnp.where(above, z_t, 0.0), axis=0, keepdims=True)
        cond = (1.0 + r * zj) > s
        kz = kz + cond.astype(f32)
        ssum = ssum + jnp.where(cond, zj, 0.0)
    tau = (ssum - 1.0) / kz
    g_t = jnp.maximum(z_t - tau, 0.0)
    g = g_t.T

    @pl.when(i == 0)
    def _():
        acc_imp[...] = jnp.zeros_like(acc_imp)
        acc_spa[...] = jnp.zeros_like(acc_spa)

    acc_imp[...] += jnp.sum(g_t, axis=1, keepdims=True)
    rn = jnp.sqrt(jnp.sum(g_t * g_t, axis=0, keepdims=True))
    acc_spa[...] += jnp.sum(rn, axis=1, keepdims=True)

    xf_t = xf_ref[...]
    o = jnp.zeros((t, K), f32) + eb2_ref[...]
    for k in range(K):
        hk = jnp.maximum(
            jnp.dot(xf_t, ew1_ref[k], preferred_element_type=f32)
            + eb1_ref[k:k + 1, :], 0.0)
        o = o + jnp.dot(hk, w2sel_ref[k * HID_MOE:(k + 1) * HID_MOE, :],
                        preferred_element_type=f32)
    go = g * o
    y_ref[...] = jnp.dot(go, jnp.ones((K, 1), f32),
                         preferred_element_type=f32)

    @pl.when(i == _NT - 1)
    def _():
        imp = acc_imp[...]
        m = jnp.sum(imp, axis=0, keepdims=True) / K
        var = jnp.sum((imp - m) ** 2, axis=0, keepdims=True) / (K - 1)
        imp_ref[...] = var / (m * m + 1e-10)
        spa_ref[...] = acc_spa[...] / B


def _tc_fused(sqlemb, xf, gw1, gb1, gw2, gb2, ew1, eb1, w2sel, eb2):
    return pl.pallas_call(
        _tc_body,
        grid=(_NT,),
        in_specs=[
            pl.BlockSpec((_TILE_B, GATE_IN), lambda i: (i, 0)),
            pl.BlockSpec((_TILE_B, EXP_IN), lambda i: (i, 0)),
            pl.BlockSpec((GATE_IN, HID_GATE), lambda i: (0, 0)),
            pl.BlockSpec((1, HID_GATE), lambda i: (0, 0)),
            pl.BlockSpec((HID_GATE, K), lambda i: (0, 0)),
            pl.BlockSpec((1, K), lambda i: (0, 0)),
            pl.BlockSpec((K, EXP_IN, HID_MOE), lambda i: (0, 0, 0)),
            pl.BlockSpec((K, HID_MOE), lambda i: (0, 0)),
            pl.BlockSpec((K * HID_MOE, K), lambda i: (0, 0)),
            pl.BlockSpec((1, K), lambda i: (0, 0)),
        ],
        out_specs=[
            pl.BlockSpec((_TILE_B, 1), lambda i: (i, 0)),
            pl.BlockSpec((1, 1), lambda i: (0, 0)),
            pl.BlockSpec((1, 1), lambda i: (0, 0)),
        ],
        out_shape=[
            jax.ShapeDtypeStruct((B, 1), jnp.float32),
            jax.ShapeDtypeStruct((1, 1), jnp.float32),
            jax.ShapeDtypeStruct((1, 1), jnp.float32),
        ],
        scratch_shapes=[
            pltpu.VMEM((K, 1), jnp.float32),
            pltpu.VMEM((1, 1), jnp.float32),
        ],
    )(sqlemb, xf, gw1, gb1, gw2, gb2, ew1, eb1, w2sel, eb2)


def kernel(x, sql, input_table, sql_table, gw1, gb1, gw2, gb2,
           ew1, eb1, ew2, eb2):
    itab_lin, stab_lin = _sc_transpose(
        input_table.T, jnp.pad(sql_table.T, ((0, 0), (0, _V2P - _V2))))
    x_rows, sql_rows = _sc_gather(
        x.reshape(-1), sql.reshape(-1), itab_lin, stab_lin)
    xf = x_rows.reshape(B, EXP_IN)
    sqlemb = sql_rows.reshape(B, GATE_IN)
    w2sel = (jnp.repeat(jnp.eye(K, dtype=jnp.float32), HID_MOE, axis=0)
             * ew2.reshape(K * HID_MOE, 1))
    y2d, imp, spa = _tc_fused(
        sqlemb, xf, gw1, gb1.reshape(1, -1),
        gw2, gb2.reshape(1, -1),
        ew1, eb1, w2sel,
        eb2.reshape(1, K))
    return y2d.reshape(B), imp[0, 0], spa[0, 0]


# speedup vs baseline: 1.0213x; 1.0213x over previous
import functools

import jax
import jax.numpy as jnp
from jax import lax
from jax.experimental import pallas as pl
from jax.experimental.pallas import tpu as pltpu
from jax.experimental.pallas import tpu_sc as plsc

NFIELD = 26
SQL_NEMB = 16
DATA_NEMB = 16
K = 16
HID_GATE = 256
HID_MOE = 512
B = 4096
GATE_IN = NFIELD * SQL_NEMB
EXP_IN = NFIELD * DATA_NEMB

_NC = 2
_NS = 16
_NW = _NC * _NS

_ROWS = B * NFIELD
_BPW = _ROWS // _NW

_TILE_B = 1024
_NT = B // _TILE_B


_V1 = 100000
_V2 = 100027
_V2P = 100352
_TCH = 3136


def _sc_transpose(itab_t, stab_t):
    mesh = plsc.VectorSubcoreMesh(core_axis_name="c", subcore_axis_name="s")

    @functools.partial(
        pl.kernel,
        out_type=(
            jax.ShapeDtypeStruct((_V1, DATA_NEMB), jnp.float32),
            jax.ShapeDtypeStruct((_V2P, SQL_NEMB), jnp.float32),
        ),
        mesh=mesh,
        compiler_params=pltpu.CompilerParams(
            use_tc_tiling_on_sc=False, needs_layout_passes=False),
        scratch_types=[
            pltpu.VMEM((16, _TCH), jnp.float32),
            pltpu.VMEM((_TCH, 16), jnp.float32),
        ],
    )
    def transpose_k(it_hbm, st_hbm, io_hbm, so_hbm, chunk_v, out_v):
        wid = lax.axis_index("s") * _NC + lax.axis_index("c")
        lanes = lax.broadcasted_iota(jnp.int32, (16,), 0)

        def one_table(src_hbm, dst_hbm, base):
            pltpu.sync_copy(src_hbm.at[:, pl.ds(base, _TCH)], chunk_v)

            def block(b, _):
                rr = b * 16
                rows = rr + lanes
                for d in range(16):
                    v = chunk_v[d, pl.ds(rr, 16)]
                    plsc.store_scatter(out_v, [rows, jnp.full((16,), d, jnp.int32)], v)
                return _

            lax.fori_loop(0, _TCH // 16, block, 0)
            pltpu.sync_copy(out_v, dst_hbm.at[pl.ds(base, _TCH), :])

        one_table(it_hbm, io_hbm, jnp.minimum(wid * _TCH, _V1 - _TCH))
        one_table(st_hbm, so_hbm, wid * _TCH)

    return transpose_k(itab_t, stab_t)


def _sc_gather(x_idx, sql_idx, input_table, sql_table):
    mesh = plsc.VectorSubcoreMesh(core_axis_name="c", subcore_axis_name="s")

    bw = B // _NW

    @functools.partial(
        pl.kernel,
        out_type=(
            jax.ShapeDtypeStruct((_ROWS, DATA_NEMB), jnp.float32),
            jax.ShapeDtypeStruct((_ROWS, SQL_NEMB), jnp.float32),
        ),
        mesh=mesh,
        compiler_params=pltpu.CompilerParams(use_tc_tiling_on_sc=False),
        scratch_types=[
            pltpu.VMEM((_BPW,), jnp.int32),
            pltpu.VMEM((_BPW, DATA_NEMB), jnp.float32),
            pltpu.SemaphoreType.DMA,
        ],
    )
    def gather_k(xi_hbm, si_hbm, itab_hbm, stab_hbm, xout_hbm, sout_hbm,
                 idx_v, rows_v, sem):
        wid = lax.axis_index("s") * _NC + lax.axis_index("c")
        base = wid * _BPW
        pltpu.sync_copy(xi_hbm.at[pl.ds(base, _BPW)], idx_v)
        pltpu.async_copy(itab_hbm.at[idx_v], rows_v, sem).wait()
        pltpu.sync_copy(rows_v, xout_hbm.at[pl.ds(base, _BPW)])
        pltpu.sync_copy(si_hbm.at[pl.ds(base, _BPW)], idx_v)
        pltpu.async_copy(stab_hbm.at[idx_v], rows_v, sem).wait()
        pltpu.sync_copy(rows_v, sout_hbm.at[pl.ds(base, _BPW)])

    return gather_k(x_idx, sql_idx, input_table, sql_table)


def _tc_body(sql_ref, xf_ref, gw1_ref, gb1_ref, gw2_ref, gb2_ref,
             ew1_ref, eb1_ref, w2sel_ref, eb2_ref,
             y_ref, imp_ref, spa_ref, acc_imp, acc_spa):
    i = pl.program_id(0)
    f32 = jnp.float32
    bf16 = jnp.bfloat16

    sql_t = sql_ref[...]
    gh = jnp.maximum(
        jnp.dot(sql_t, gw1_ref[...], preferred_element_type=f32) + gb1_ref[...],
        0.0)
    logits = (jnp.dot(gh, gw2_ref[...],
                      preferred_element_type=f32) + gb2_ref[...])

    t = logits.shape[0]
    z_t = logits.T
    rowid = lax.broadcasted_iota(jnp.int32, (K, t), 0)
    kz = jnp.zeros((1, t), f32)
    ssum = jnp.zeros((1, t), f32)
    for j in range(K):
        zj = z_t[j:j + 1, :]
        above = jnp.logical_or(
            z_t > zj, jnp.logical_and(z_t == zj, rowid < j))
        r = 1.0 + jnp.sum(above.astype(f32), axis=0, keepdims=True)
        s = zj + jnp.sum(jnp.where(above, z_t, 0.0), axis=0, keepdims=True)
        cond = (1.0 + r * zj) > s
        kz = kz + cond.astype(f32)
        ssum = ssum + jnp.where(cond, zj, 0.0)
    tau = (ssum - 1.0) / kz
    g_t = jnp.maximum(z_t - tau, 0.0)
    g = g_t.T

    @pl.when(i == 0)
    def _():
        acc_imp[...] = jnp.zeros_like(acc_imp)
        acc_spa[...] = jnp.zeros_like(acc_spa)

    acc_imp[...] += jnp.sum(g_t, axis=1, keepdims=True)
    rn = jnp.sqrt(jnp.sum(g_t * g_t, axis=0, keepdims=True))
    acc_spa[...] += jnp.sum(rn, axis=1, keepdims=True)

    xf_t = xf_ref[...]
    o = jnp.zeros((t, K), f32) + eb2_ref[...]
    for k in range(K):
        hk = jnp.maximum(
            jnp.dot(xf_t, ew1_ref[k], preferred_element_type=f32)
            + eb1_ref[k:k + 1, :], 0.0)
        o = o + jnp.dot(hk, w2sel_ref[k * HID_MOE:(k + 1) * HID_MOE, :],
                        preferred_element_type=f32)
    go = g * o
    y_ref[...] = jnp.dot(go, jnp.ones((K, 1), f32),
                         preferred_element_type=f32)

    @pl.when(i == _NT - 1)
    def _():
        imp = acc_imp[...]
        m = jnp.sum(imp, axis=0, keepdims=True) / K
        var = jnp.sum((imp - m) ** 2, axis=0, keepdims=True) / (K - 1)
        imp_ref[...] = var / (m * m + 1e-10)
        spa_ref[...] = acc_spa[...] / B


def _tc_fused(sqlemb, xf, gw1, gb1, gw2, gb2, ew1, eb1, w2sel, eb2):
    return pl.pallas_call(
        _tc_body,
        grid=(_NT,),
        in_specs=[
            pl.BlockSpec((_TILE_B, GATE_IN), lambda i: (i, 0)),
            pl.BlockSpec((_TILE_B, EXP_IN), lambda i: (i, 0)),
            pl.BlockSpec((GATE_IN, HID_GATE), lambda i: (0, 0)),
            pl.BlockSpec((1, HID_GATE), lambda i: (0, 0)),
            pl.BlockSpec((HID_GATE, K), lambda i: (0, 0)),
            pl.BlockSpec((1, K), lambda i: (0, 0)),
            pl.BlockSpec((K, EXP_IN, HID_MOE), lambda i: (0, 0, 0)),
            pl.BlockSpec((K, HID_MOE), lambda i: (0, 0)),
            pl.BlockSpec((K * HID_MOE, K), lambda i: (0, 0)),
            pl.BlockSpec((1, K), lambda i: (0, 0)),
        ],
        out_specs=[
            pl.BlockSpec((_TILE_B, 1), lambda i: (i, 0)),
            pl.BlockSpec((1, 1), lambda i: (0, 0)),
            pl.BlockSpec((1, 1), lambda i: (0, 0)),
        ],
        out_shape=[
            jax.ShapeDtypeStruct((B, 1), jnp.float32),
            jax.ShapeDtypeStruct((1, 1), jnp.float32),
            jax.ShapeDtypeStruct((1, 1), jnp.float32),
        ],
        scratch_shapes=[
            pltpu.VMEM((K, 1), jnp.float32),
            pltpu.VMEM((1, 1), jnp.float32),
        ],
    )(sqlemb, xf, gw1, gb1, gw2, gb2, ew1, eb1, w2sel, eb2)


def kernel(x, sql, input_table, sql_table, gw1, gb1, gw2, gb2,
           ew1, eb1, ew2, eb2):
    itab_lin, stab_lin = _sc_transpose(
        input_table.T, jnp.pad(sql_table.T, ((0, 0), (0, _V2P - _V2))))
    x_rows, sql_rows = _sc_gather(
        x.reshape(-1), sql.reshape(-1), itab_lin, stab_lin)
    xf = x_rows.reshape(B, EXP_IN)
    sqlemb = sql_rows.reshape(B, GATE_IN)
    w2sel = (jnp.repeat(jnp.eye(K, dtype=jnp.float32), HID_MOE, axis=0)
             * ew2.reshape(K * HID_MOE, 1))
    y2d, imp, spa = _tc_fused(
        sqlemb, xf, gw1, gb1.reshape(1, -1),
        gw2, gb2.reshape(1, -1),
        ew1, eb1, w2sel,
        eb2.reshape(1, K))
    return y2d.reshape(B), imp[0, 0], spa[0, 0]
